# baseline (speedup 1.0000x reference)
import functools

import jax
import jax.numpy as jnp
from jax import lax
from jax.experimental import pallas as pl
from jax.experimental.pallas import tpu as pltpu
from jax.experimental.pallas import tpu_sc as plsc

DEPTH = 4
T = 1000
W = 128

_SC_CHUNK = 2000
_SC_ACTIVE = 25
_SC_WORKERS = 32


def _make_cnt_sc(N, B):
    mesh = plsc.VectorSubcoreMesh(core_axis_name="c", subcore_axis_name="s",
                                  num_cores=2, num_subcores=16)

    @functools.partial(
        pl.kernel,
        out_type=jax.ShapeDtypeStruct((_SC_WORKERS, B), jnp.float32),
        mesh=mesh,
        compiler_params=pltpu.CompilerParams(needs_layout_passes=False),
        scratch_types=[
            pltpu.VMEM((_SC_CHUNK,), jnp.int32),
            pltpu.VMEM((B,), jnp.float32),
        ],
    )
    def cnt_sc(ids_hbm, out_hbm, ids_v, acc_v):
        nc = mesh.num_cores
        wid = lax.axis_index("s") * nc + lax.axis_index("c")
        zero16 = jnp.zeros((16,), jnp.float32)
        for j in range(B // 16):
            acc_v[pl.ds(j * 16, 16)] = zero16

        @pl.when(wid < _SC_ACTIVE)
        def _count():
            pltpu.sync_copy(ids_hbm.at[pl.ds(wid * _SC_CHUNK, _SC_CHUNK)],
                            ids_v)
            ones16 = jnp.ones((16,), jnp.float32)

            def step(k, _):
                idx = ids_v[pl.ds(k * 16, 16)]
                plsc.addupdate_scatter(acc_v, [idx], ones16)
                return 0

            lax.fori_loop(0, _SC_CHUNK // 16, step, 0)

        pltpu.sync_copy(acc_v, out_hbm.at[wid])

    return cnt_sc


def _main_body(ids_ref, nodes_ref, Wv_ref, bv_ref, Wacat_ref, bacat_ref,
               Walb_ref, gk_ref, grk_ref, gb_ref, out_ref,
               sumhv_ref, wsum_ref, scal_ref):
    pid = pl.program_id(0)

    @pl.when(pid == 0)
    def _init():
        sumhv_ref[...] = jnp.zeros_like(sumhv_ref)
        wsum_ref[...] = jnp.zeros_like(wsum_ref)
        scal_ref[...] = jnp.zeros_like(scal_ref)

    ids_row = ids_ref[0]
    lo = ids_ref[0, 0, 0]
    hi = ids_ref[0, 0, T - 1]

    hv = jnp.maximum(
        jnp.dot(nodes_ref[...].astype(jnp.bfloat16), Wv_ref[...],
                preferred_element_type=jnp.float32) + bv_ref[...], 0.0)
    hvb = hv.astype(jnp.bfloat16)
    att = jnp.maximum(
        jnp.dot(hvb, Wacat_ref[...],
                preferred_element_type=jnp.float32) + bacat_ref[...], 0.0)
    attb = att.astype(jnp.bfloat16)
    s4 = jnp.dot(hv, Walb_ref[...], preferred_element_type=jnp.float32)
    e4 = jnp.exp(s4)
    e4b = e4.astype(jnp.bfloat16)
    e4tb = e4b.T
    eaug = jnp.concatenate(
        [e4b, jnp.ones((T, 1), jnp.bfloat16)], axis=1)

    base0 = (lo // W) * W
    nchunks = hi // W - lo // W + 1

    def chunk(c, _):
        base = base0 + c * W
        win = base + lax.broadcasted_iota(jnp.int32, (W, T), 0)
        M = (ids_row == win).astype(jnp.bfloat16)
        sumhv_ref[pl.ds(base, W), :] += jnp.dot(
            M, hvb, preferred_element_type=jnp.float32)
        scal_ref[pl.ds(base, W), 0:DEPTH + 1] += jnp.dot(
            M, eaug, preferred_element_type=jnp.float32)
        for i in range(DEPTH):
            Me = M * e4tb[i:i + 1, :]
            wsum_ref[i, pl.ds(base, W), :] += jnp.dot(
                Me, attb[:, i * 256:(i + 1) * 256],
                preferred_element_type=jnp.float32)
        return 0

    lax.fori_loop(0, nchunks, chunk, 0)

    @pl.when(pid == pl.num_programs(0) - 1)
    def _finish():
        cnt = scal_ref[:, DEPTH:DEPTH + 1]
        hm = sumhv_ref[...] / jnp.maximum(cnt, 1.0)
        for i in range(DEPTH):
            esum = scal_ref[:, i:i + 1]
            mm = wsum_ref[i] / (esum + 1e-9)
            mm = jnp.where(mm > 0.0, mm,
                           jnp.exp(jnp.minimum(mm, 0.0)) - 1.0)
            mx = jnp.dot(mm.astype(jnp.bfloat16), gk_ref[...],
                         preferred_element_type=jnp.float32) + gb_ref[0:1, :]
            mh = jnp.dot(hm.astype(jnp.bfloat16), grk_ref[...],
                         preferred_element_type=jnp.float32) + gb_ref[1:2, :]
            z = jax.nn.sigmoid(mx[:, 0:256] + mh[:, 0:256])
            r = jax.nn.sigmoid(mx[:, 256:512] + mh[:, 256:512])
            hh = jnp.tanh(mx[:, 512:768] + r * mh[:, 512:768])
            hm = jnp.maximum(z * hm + (1.0 - z) * hh, 0.0)
        out_ref[...] = hm


def kernel(count_nodes, nodes, batch_id, Wv, bv, Wa, ba, Wal, bal, gru_k,
           gru_rk, gru_b):
    N, F = nodes.shape
    B = count_nodes.shape[0]
    U = Wv.shape[1]
    nb = N // T

    ids3 = batch_id.astype(jnp.int32).reshape(nb, 1, T)
    Wv_b = Wv.astype(jnp.bfloat16)
    Wacat = jnp.transpose(Wa, (1, 0, 2)).reshape(F, DEPTH * U).astype(
        jnp.bfloat16)
    bacat = ba.reshape(1, DEPTH * U)
    Walb = jnp.transpose(Wal[:, U:, 0])
    bv_row = bv.reshape(1, U)

    out = pl.pallas_call(
        _main_body,
        grid=(nb,),
        in_specs=[
            pl.BlockSpec((1, 1, T), lambda i: (i, 0, 0)),
            pl.BlockSpec((T, F), lambda i: (i, 0)),
            pl.BlockSpec((F, U), lambda i: (0, 0)),
            pl.BlockSpec((1, U), lambda i: (0, 0)),
            pl.BlockSpec((F, DEPTH * U), lambda i: (0, 0)),
            pl.BlockSpec((1, DEPTH * U), lambda i: (0, 0)),
            pl.BlockSpec((F, DEPTH), lambda i: (0, 0)),
            pl.BlockSpec((U, 3 * U), lambda i: (0, 0)),
            pl.BlockSpec((U, 3 * U), lambda i: (0, 0)),
            pl.BlockSpec((2, 3 * U), lambda i: (0, 0)),
        ],
        out_specs=pl.BlockSpec((B, U), lambda i: (0, 0)),
        out_shape=jax.ShapeDtypeStruct((B, U), jnp.float32),
        scratch_shapes=[
            pltpu.VMEM((B, U), jnp.float32),
            pltpu.VMEM((DEPTH, B, U), jnp.float32),
            pltpu.VMEM((B, 8), jnp.float32),
        ],
    )(ids3, nodes, Wv_b, bv_row, Wacat, bacat, Walb,
      gru_k.astype(jnp.bfloat16), gru_rk.astype(jnp.bfloat16), gru_b)
    return out


# speedup vs baseline: 1.0409x; 1.0409x over previous
import jax
import jax.numpy as jnp
from jax import lax
from jax.experimental import pallas as pl
from jax.experimental.pallas import tpu as pltpu

DEPTH = 4
T = 1000
W = 128


def _main_body(ids_ref, nodes_ref, Wv_ref, bv_ref, Wacat_ref, bacat_ref,
               Walb_ref, gk_ref, grk_ref, gb_ref, out_ref,
               sumhv_ref, wsum_ref, scal_ref):
    pid = pl.program_id(0)

    @pl.when(pid == 0)
    def _init():
        sumhv_ref[...] = jnp.zeros_like(sumhv_ref)
        wsum_ref[...] = jnp.zeros_like(wsum_ref)
        scal_ref[...] = jnp.zeros_like(scal_ref)

    ids_row = ids_ref[0]
    lo = ids_ref[0, 0, 0]
    hi = ids_ref[0, 0, T - 1]

    hv = jnp.maximum(
        jnp.dot(nodes_ref[...].astype(jnp.bfloat16), Wv_ref[...],
                preferred_element_type=jnp.float32) + bv_ref[...], 0.0)
    hvb = hv.astype(jnp.bfloat16)
    att = jnp.maximum(
        jnp.dot(hvb, Wacat_ref[...],
                preferred_element_type=jnp.float32) + bacat_ref[...], 0.0)
    attb = att.astype(jnp.bfloat16)
    s4 = jnp.dot(hv, Walb_ref[...], preferred_element_type=jnp.float32)
    e4 = jnp.exp(s4)
    e4b = e4.astype(jnp.bfloat16)
    e4tb = e4b.T
    eaug = jnp.concatenate(
        [e4b, jnp.ones((T, 1), jnp.bfloat16)], axis=1)

    base0 = (lo // W) * W
    nchunks = hi // W - lo // W + 1

    def chunk(c, _):
        base = base0 + c * W
        win = base + lax.broadcasted_iota(jnp.int32, (W, T), 0)
        M = (ids_row == win).astype(jnp.bfloat16)
        sumhv_ref[pl.ds(base, W), :] += jnp.dot(
            M, hvb, preferred_element_type=jnp.float32)
        scal_ref[pl.ds(base, W), 0:DEPTH + 1] += jnp.dot(
            M, eaug, preferred_element_type=jnp.float32)
        for i in range(DEPTH):
            Me = M * e4tb[i:i + 1, :]
            wsum_ref[i, pl.ds(base, W), :] += jnp.dot(
                Me, attb[:, i * 256:(i + 1) * 256],
                preferred_element_type=jnp.float32)
        return 0

    lax.fori_loop(0, nchunks, chunk, 0)

    @pl.when(pid == pl.num_programs(0) - 1)
    def _finish():
        cnt = scal_ref[:, DEPTH:DEPTH + 1]
        hm = sumhv_ref[...] / jnp.maximum(cnt, 1.0)
        for i in range(DEPTH):
            esum = scal_ref[:, i:i + 1]
            mm = wsum_ref[i] / (esum + 1e-9)
            mm = jnp.where(mm > 0.0, mm,
                           jnp.exp(jnp.minimum(mm, 0.0)) - 1.0)
            mx = jnp.dot(mm, gk_ref[...],
                         preferred_element_type=jnp.float32) + gb_ref[0:1, :]
            mh = jnp.dot(hm, grk_ref[...],
                         preferred_element_type=jnp.float32) + gb_ref[1:2, :]
            z = jax.nn.sigmoid(mx[:, 0:256] + mh[:, 0:256])
            r = jax.nn.sigmoid(mx[:, 256:512] + mh[:, 256:512])
            hh = jnp.tanh(mx[:, 512:768] + r * mh[:, 512:768])
            hm = jnp.maximum(z * hm + (1.0 - z) * hh, 0.0)
        out_ref[...] = hm


def kernel(count_nodes, nodes, batch_id, Wv, bv, Wa, ba, Wal, bal, gru_k,
           gru_rk, gru_b):
    N, F = nodes.shape
    B = count_nodes.shape[0]
    U = Wv.shape[1]
    nb = N // T

    ids3 = batch_id.astype(jnp.int32).reshape(nb, 1, T)
    Wv_b = Wv.astype(jnp.bfloat16)
    Wacat = jnp.transpose(Wa, (1, 0, 2)).reshape(F, DEPTH * U).astype(
        jnp.bfloat16)
    bacat = ba.reshape(1, DEPTH * U)
    Walb = jnp.transpose(Wal[:, U:, 0])
    bv_row = bv.reshape(1, U)

    out = pl.pallas_call(
        _main_body,
        grid=(nb,),
        in_specs=[
            pl.BlockSpec((1, 1, T), lambda i: (i, 0, 0)),
            pl.BlockSpec((T, F), lambda i: (i, 0)),
            pl.BlockSpec((F, U), lambda i: (0, 0)),
            pl.BlockSpec((1, U), lambda i: (0, 0)),
            pl.BlockSpec((F, DEPTH * U), lambda i: (0, 0)),
            pl.BlockSpec((1, DEPTH * U), lambda i: (0, 0)),
            pl.BlockSpec((F, DEPTH), lambda i: (0, 0)),
            pl.BlockSpec((U, 3 * U), lambda i: (0, 0)),
            pl.BlockSpec((U, 3 * U), lambda i: (0, 0)),
            pl.BlockSpec((2, 3 * U), lambda i: (0, 0)),
        ],
        out_specs=pl.BlockSpec((B, U), lambda i: (0, 0)),
        out_shape=jax.ShapeDtypeStruct((B, U), jnp.float32),
        scratch_shapes=[
            pltpu.VMEM((B, U), jnp.float32),
            pltpu.VMEM((DEPTH, B, U), jnp.float32),
            pltpu.VMEM((B, 8), jnp.float32),
        ],
    )(ids3, nodes, Wv_b, bv_row, Wacat, bacat, Walb, gru_k, gru_rk, gru_b)
    return out
